# baseline (speedup 1.0000x reference)
import functools

import jax
import jax.numpy as jnp
from jax import lax
from jax.experimental import pallas as pl
from jax.experimental.pallas import tpu as pltpu
from jax.experimental.pallas import tpu_sc as plsc

NPT = 320
CH = 128
NTILES = 32
L = 16


def _segmax_body(W, feat_hbm, src_hbm, dst_hbm, bnd_hbm, out_hbm,
                 acc_v, idx0, dst0, rows0, idx1, dst1, rows1, bnd_v,
                 sem0, sem1):
    cid = lax.axis_index("c")
    sid = lax.axis_index("s")
    wid = sid * 2 + cid

    pltpu.sync_copy(bnd_hbm, bnd_v)
    lo = bnd_v[pl.ds(wid, L)][0]
    hi = bnd_v[pl.ds(wid + 1, L)][0]
    base0 = (lo // 8) * 8
    nchunks = (hi - base0 + (CH - 1)) // CH
    node_lo = wid * NPT

    neg = jnp.full((L,), -jnp.inf, jnp.float32)

    def _init(r, _):
        for cb in range(W // L):
            acc_v[r, pl.ds(cb * L, L)] = neg
        return 0
    lax.fori_loop(0, NPT + 1, _init, 0)

    bufs = [(idx0, dst0, rows0, sem0), (idx1, dst1, rows1, sem1)]

    def _fetch(c, b):
        idx_v, dst_v, rows_v, sem = bufs[b]
        eb = pl.multiple_of(base0 + c * CH, 8)
        pltpu.sync_copy(src_hbm.at[pl.ds(eb, CH)], idx_v)
        pltpu.sync_copy(dst_hbm.at[pl.ds(eb, CH)], dst_v)
        pltpu.async_copy(feat_hbm.at[idx_v], rows_v, sem)

    nblk = W // L
    neginf = jnp.full((L,), -jnp.inf, jnp.float32)

    def _process(b, carry):
        idx_v, dst_v, rows_v, sem = bufs[b]
        pltpu.make_async_copy(feat_hbm.at[idx_v], rows_v, sem).wait()

        def _grp(g, car):
            rd = car[0]
            ms = list(car[1:])
            dvec = dst_v[pl.ds(g * L, L)]
            for j in range(L):
                d = dvec[j]
                sw = d != rd
                dl = d - node_lo
                ok = (dl >= 0) & (dl < NPT)
                dlc = jnp.where(ok, dl, NPT)
                r = g * L + j
                for cb in range(nblk):
                    s = pl.ds(cb * L, L)
                    prev = jnp.where(sw, neginf, ms[cb])
                    ms[cb] = jnp.maximum(prev, rows_v[r, s])
                    acc_v[dlc, s] = ms[cb]
                rd = d
            return (rd, *ms)

        return lax.fori_loop(0, CH // L, _grp, carry)

    _fetch(0, 0)
    _fetch(1, 1)

    def _pair(c2, carry):
        for b in range(2):
            c = c2 * 2 + b
            carry = _process(b, carry)
            _fetch(c + 2, b)
        return carry

    carry0 = (jnp.int32(-1),) + (neginf,) * nblk
    lax.fori_loop(0, (nchunks + 1) // 2, _pair, carry0)
    pltpu.make_async_copy(feat_hbm.at[idx0], rows0, sem0).wait()
    pltpu.make_async_copy(feat_hbm.at[idx1], rows1, sem1).wait()

    def _fix(r, _):
        for cb in range(W // L):
            s = pl.ds(cb * L, L)
            v = acc_v[r, s]
            acc_v[r, s] = jnp.where(v == -jnp.inf, 0.0, v)
        return 0
    lax.fori_loop(0, NPT, _fix, 0)

    pltpu.sync_copy(acc_v.at[pl.ds(0, NPT)], out_hbm.at[pl.ds(wid * NPT, NPT)])


@functools.cache
def _segmax_call(W):
    mesh = plsc.VectorSubcoreMesh(core_axis_name="c", subcore_axis_name="s",
                                  num_cores=2, num_subcores=16)
    return pl.kernel(
        functools.partial(_segmax_body, W),
        out_type=jax.ShapeDtypeStruct((NTILES * NPT, W), jnp.float32),
        mesh=mesh,
        compiler_params=pltpu.CompilerParams(use_tc_tiling_on_sc=False),
        scratch_types=[
            pltpu.VMEM((NPT + 1, W), jnp.float32),
            pltpu.VMEM((CH,), jnp.int32),
            pltpu.VMEM((CH,), jnp.int32),
            pltpu.VMEM((CH, W), jnp.float32),
            pltpu.VMEM((CH,), jnp.int32),
            pltpu.VMEM((CH,), jnp.int32),
            pltpu.VMEM((CH, W), jnp.float32),
            pltpu.VMEM((64,), jnp.int32),
            pltpu.SemaphoreType.DMA,
            pltpu.SemaphoreType.DMA,
        ],
    )


def _segmax(feat, srcs, dsts, bnd, n):
    W = feat.shape[1]
    return _segmax_call(W)(feat, srcs, dsts, bnd)[:n]


def _dense_body(nin, relu, *refs):
    out = refs[-1]
    b = refs[-2][...]
    acc = jnp.zeros(out.shape, jnp.float32)
    for i in range(nin):
        x = refs[2 * i][...]
        w = refs[2 * i + 1][...]
        acc = acc + jax.lax.dot(x, w, preferred_element_type=jnp.float32)
    acc = acc + b
    if relu:
        acc = jnp.maximum(acc, 0.0)
    out[...] = acc


def _dense(inputs, b, relu):
    n = inputs[0][0].shape[0]
    do = b.shape[0]
    grid = 10
    r = n // grid
    specs = []
    args = []
    for x, wt in inputs:
        di = x.shape[1]
        specs.append(pl.BlockSpec((r, di), lambda i: (i, 0)))
        specs.append(pl.BlockSpec((di, do), lambda i: (0, 0)))
        args.extend([x, wt])
    specs.append(pl.BlockSpec((1, do), lambda i: (0, 0)))
    args.append(b.reshape(1, do))
    return pl.pallas_call(
        functools.partial(_dense_body, len(inputs), relu),
        grid=(grid,),
        in_specs=specs,
        out_specs=pl.BlockSpec((r, do), lambda i: (i, 0)),
        out_shape=jax.ShapeDtypeStruct((n, do), jnp.float32),
    )(*args)


def _head_body(rm_ref, j2_ref, w_ref, b_ref, ls_ref, jm_ref, lab_ref):
    rm = rm_ref[...]
    m = jnp.max(rm, axis=1, keepdims=True)
    e = jnp.exp(rm - m)
    s = jnp.sum(e, axis=1, keepdims=True)
    ls_ref[...] = rm - m - jnp.log(s)
    r0 = rm[:, 0:1]
    r1 = rm[:, 1:2]
    r2 = rm[:, 2:3]
    lab_ref[...] = jnp.where((r2 > r0) & (r2 > r1), 2.0, 0.0)
    j2 = j2_ref[...]
    jm_ref[...] = (jax.lax.dot(j2, w_ref[...],
                               preferred_element_type=jnp.float32)
                   + b_ref[...])


def _head(rm, j2, j3_Wt, j3_b):
    n = rm.shape[0]
    grid = 10
    r = n // grid
    return pl.pallas_call(
        _head_body,
        grid=(grid,),
        in_specs=[
            pl.BlockSpec((r, 3), lambda i: (i, 0)),
            pl.BlockSpec((r, 32), lambda i: (i, 0)),
            pl.BlockSpec((32, 2), lambda i: (0, 0)),
            pl.BlockSpec((1, 2), lambda i: (0, 0)),
        ],
        out_specs=[
            pl.BlockSpec((r, 3), lambda i: (i, 0)),
            pl.BlockSpec((r, 2), lambda i: (i, 0)),
            pl.BlockSpec((r, 1), lambda i: (i, 0)),
        ],
        out_shape=[
            jax.ShapeDtypeStruct((n, 3), jnp.float32),
            jax.ShapeDtypeStruct((n, 2), jnp.float32),
            jax.ShapeDtypeStruct((n, 1), jnp.float32),
        ],
    )(rm, j2, j3_Wt, j3_b.reshape(1, 2))


def kernel(x, edge_index, rm1_Wl, rm1_bl, rm1_Wr, rm2_Wl, rm2_bl, rm2_Wr,
           rm3_Wl, rm3_bl, rm3_Wr, rm4_Wl, rm4_bl, rm4_Wr, sh_Wl, sh_bl,
           sh_Wr, j1_Wl, j1_bl, j1_Wr, j2_Wl, j2_bl, j2_Wr, j3_W, j3_b):
    n = x.shape[0]
    e = edge_index.shape[1]
    src, dst = edge_index[0], edge_index[1]

    order = jnp.argsort(dst)
    dsts = dst[order]
    srcs = src[order]
    bnd = jnp.searchsorted(dsts, jnp.arange(NTILES + 1, dtype=jnp.int32) * NPT
                           ).astype(jnp.int32)
    bnd = jnp.pad(bnd, (0, 64 - (NTILES + 1)), constant_values=e)
    srcs = jnp.pad(srcs, (0, 4 * CH))
    dsts = jnp.pad(dsts, (0, 4 * CH), constant_values=NTILES * NPT + 1)

    sm = lambda f: _segmax(f, srcs, dsts, bnd, n)

    agg_x = sm(x)
    rm1 = _dense([(agg_x, rm1_Wl.T), (x, rm1_Wr.T)], rm1_bl, True)
    rm2 = _dense([(sm(rm1), rm2_Wl.T), (rm1, rm2_Wr.T)], rm2_bl, True)
    rm3 = _dense([(sm(rm2), rm3_Wl.T), (rm2, rm3_Wr.T)], rm3_bl, True)
    rm4 = _dense([(sm(rm3), rm4_Wl.T), (rm3, rm4_Wr.T)], rm4_bl, False)

    rm4p = jnp.pad(rm4, ((0, 0), (0, 13)))
    agg_rm4 = sm(rm4p)[:, :3]
    h = _dense(
        [(agg_x, sh_Wl[:, :128].T), (agg_rm4, sh_Wl[:, 128:].T),
         (x, sh_Wr[:, :128].T), (rm4, sh_Wr[:, 128:].T)],
        sh_bl, True)

    j1 = _dense([(sm(h), j1_Wl.T), (h, j1_Wr.T)], j1_bl, True)
    j2 = _dense([(sm(j1), j2_Wl.T), (j1, j2_Wr.T)], j2_bl, True)

    rm_ls, jm, lab = _head(rm4, j2, j3_W.T, j3_b)

    lab2 = lab.reshape(n // 2, 2)
    labcat = jnp.concatenate([lab2, lab2], axis=0)
    j = (jm * labcat).reshape(-1)
    return (rm_ls, j)


# speedup vs baseline: 1.0537x; 1.0537x over previous
import functools

import jax
import jax.numpy as jnp
from jax import lax
from jax.experimental import pallas as pl
from jax.experimental.pallas import tpu as pltpu
from jax.experimental.pallas import tpu_sc as plsc

NPT = 320
CH = 128
NTILES = 32
L = 16


def _segmax_body(W, feat_hbm, src_hbm, dst_hbm, bnd_hbm, out_hbm,
                 acc_v, idx0, dst0, rows0, idx1, dst1, rows1, bnd_v,
                 sem0, sem1):
    cid = lax.axis_index("c")
    sid = lax.axis_index("s")
    wid = sid * 2 + cid

    pltpu.sync_copy(bnd_hbm, bnd_v)
    lo = bnd_v[pl.ds(wid, L)][0]
    hi = bnd_v[pl.ds(wid + 1, L)][0]
    base0 = (lo // 8) * 8
    nchunks = (hi - base0 + (CH - 1)) // CH
    node_lo = wid * NPT

    neg = jnp.full((L,), -jnp.inf, jnp.float32)

    def _init(r, _):
        for cb in range(W // L):
            acc_v[r, pl.ds(cb * L, L)] = neg
        return 0
    lax.fori_loop(0, NPT + 1, _init, 0)

    bufs = [(idx0, dst0, rows0, sem0), (idx1, dst1, rows1, sem1)]

    def _fetch(c, b):
        idx_v, dst_v, rows_v, sem = bufs[b]
        eb = pl.multiple_of(base0 + c * CH, 8)
        pltpu.sync_copy(src_hbm.at[pl.ds(eb, CH)], idx_v)
        pltpu.sync_copy(dst_hbm.at[pl.ds(eb, CH)], dst_v)
        pltpu.async_copy(feat_hbm.at[idx_v], rows_v, sem)

    def _process(b):
        idx_v, dst_v, rows_v, sem = bufs[b]
        pltpu.make_async_copy(feat_hbm.at[idx_v], rows_v, sem).wait()

        def _grp(g, _):
            dvec = dst_v[pl.ds(g * L, L)]
            for j in range(L):
                dl = dvec[j] - node_lo
                ok = (dl >= 0) & (dl < NPT)
                dlc = jnp.where(ok, dl, NPT)
                r = g * L + j
                for cb in range(W // L):
                    s = pl.ds(cb * L, L)
                    acc_v[dlc, s] = jnp.maximum(acc_v[dlc, s],
                                                rows_v[r, s])
            return 0

        lax.fori_loop(0, CH // L, _grp, 0)

    _fetch(0, 0)
    _fetch(1, 1)

    def _pair(c2, _):
        for b in range(2):
            c = c2 * 2 + b
            _process(b)
            _fetch(c + 2, b)
        return 0

    lax.fori_loop(0, (nchunks + 1) // 2, _pair, 0)
    pltpu.make_async_copy(feat_hbm.at[idx0], rows0, sem0).wait()
    pltpu.make_async_copy(feat_hbm.at[idx1], rows1, sem1).wait()

    def _fix(r, _):
        for cb in range(W // L):
            s = pl.ds(cb * L, L)
            v = acc_v[r, s]
            acc_v[r, s] = jnp.where(v == -jnp.inf, 0.0, v)
        return 0
    lax.fori_loop(0, NPT, _fix, 0)

    pltpu.sync_copy(acc_v.at[pl.ds(0, NPT)], out_hbm.at[pl.ds(wid * NPT, NPT)])


@functools.cache
def _segmax_call(W):
    mesh = plsc.VectorSubcoreMesh(core_axis_name="c", subcore_axis_name="s",
                                  num_cores=2, num_subcores=16)
    return pl.kernel(
        functools.partial(_segmax_body, W),
        out_type=jax.ShapeDtypeStruct((NTILES * NPT, W), jnp.float32),
        mesh=mesh,
        compiler_params=pltpu.CompilerParams(use_tc_tiling_on_sc=False),
        scratch_types=[
            pltpu.VMEM((NPT + 1, W), jnp.float32),
            pltpu.VMEM((CH,), jnp.int32),
            pltpu.VMEM((CH,), jnp.int32),
            pltpu.VMEM((CH, W), jnp.float32),
            pltpu.VMEM((CH,), jnp.int32),
            pltpu.VMEM((CH,), jnp.int32),
            pltpu.VMEM((CH, W), jnp.float32),
            pltpu.VMEM((64,), jnp.int32),
            pltpu.SemaphoreType.DMA,
            pltpu.SemaphoreType.DMA,
        ],
    )


def _segmax(feat, srcs, dsts, bnd, n):
    W = feat.shape[1]
    return _segmax_call(W)(feat, srcs, dsts, bnd)[:n]


def _dense_body(nin, relu, *refs):
    out = refs[-1]
    b = refs[-2][...]
    acc = jnp.zeros(out.shape, jnp.float32)
    for i in range(nin):
        x = refs[2 * i][...]
        w = refs[2 * i + 1][...]
        acc = acc + jax.lax.dot(x, w, preferred_element_type=jnp.float32)
    acc = acc + b
    if relu:
        acc = jnp.maximum(acc, 0.0)
    out[...] = acc


def _dense(inputs, b, relu):
    n = inputs[0][0].shape[0]
    do = b.shape[0]
    grid = 10
    r = n // grid
    specs = []
    args = []
    for x, wt in inputs:
        di = x.shape[1]
        specs.append(pl.BlockSpec((r, di), lambda i: (i, 0)))
        specs.append(pl.BlockSpec((di, do), lambda i: (0, 0)))
        args.extend([x, wt])
    specs.append(pl.BlockSpec((1, do), lambda i: (0, 0)))
    args.append(b.reshape(1, do))
    return pl.pallas_call(
        functools.partial(_dense_body, len(inputs), relu),
        grid=(grid,),
        in_specs=specs,
        out_specs=pl.BlockSpec((r, do), lambda i: (i, 0)),
        out_shape=jax.ShapeDtypeStruct((n, do), jnp.float32),
    )(*args)


def _head_body(rm_ref, j2_ref, w_ref, b_ref, ls_ref, jm_ref, lab_ref):
    rm = rm_ref[...]
    m = jnp.max(rm, axis=1, keepdims=True)
    e = jnp.exp(rm - m)
    s = jnp.sum(e, axis=1, keepdims=True)
    ls_ref[...] = rm - m - jnp.log(s)
    r0 = rm[:, 0:1]
    r1 = rm[:, 1:2]
    r2 = rm[:, 2:3]
    lab_ref[...] = jnp.where((r2 > r0) & (r2 > r1), 2.0, 0.0)
    j2 = j2_ref[...]
    jm_ref[...] = (jax.lax.dot(j2, w_ref[...],
                               preferred_element_type=jnp.float32)
                   + b_ref[...])


def _head(rm, j2, j3_Wt, j3_b):
    n = rm.shape[0]
    grid = 10
    r = n // grid
    return pl.pallas_call(
        _head_body,
        grid=(grid,),
        in_specs=[
            pl.BlockSpec((r, 3), lambda i: (i, 0)),
            pl.BlockSpec((r, 32), lambda i: (i, 0)),
            pl.BlockSpec((32, 2), lambda i: (0, 0)),
            pl.BlockSpec((1, 2), lambda i: (0, 0)),
        ],
        out_specs=[
            pl.BlockSpec((r, 3), lambda i: (i, 0)),
            pl.BlockSpec((r, 2), lambda i: (i, 0)),
            pl.BlockSpec((r, 1), lambda i: (i, 0)),
        ],
        out_shape=[
            jax.ShapeDtypeStruct((n, 3), jnp.float32),
            jax.ShapeDtypeStruct((n, 2), jnp.float32),
            jax.ShapeDtypeStruct((n, 1), jnp.float32),
        ],
    )(rm, j2, j3_Wt, j3_b.reshape(1, 2))


def kernel(x, edge_index, rm1_Wl, rm1_bl, rm1_Wr, rm2_Wl, rm2_bl, rm2_Wr,
           rm3_Wl, rm3_bl, rm3_Wr, rm4_Wl, rm4_bl, rm4_Wr, sh_Wl, sh_bl,
           sh_Wr, j1_Wl, j1_bl, j1_Wr, j2_Wl, j2_bl, j2_Wr, j3_W, j3_b):
    n = x.shape[0]
    e = edge_index.shape[1]
    src, dst = edge_index[0], edge_index[1]

    order = jnp.argsort(dst)
    dsts = dst[order]
    srcs = src[order]
    bnd = jnp.searchsorted(dsts, jnp.arange(NTILES + 1, dtype=jnp.int32) * NPT
                           ).astype(jnp.int32)
    bnd = jnp.pad(bnd, (0, 64 - (NTILES + 1)), constant_values=e)
    srcs = jnp.pad(srcs, (0, 4 * CH))
    dsts = jnp.pad(dsts, (0, 4 * CH), constant_values=NTILES * NPT + 1)

    sm = lambda f: _segmax(f, srcs, dsts, bnd, n)

    agg_x = sm(x)
    rm1 = _dense([(agg_x, rm1_Wl.T), (x, rm1_Wr.T)], rm1_bl, True)
    rm2 = _dense([(sm(rm1), rm2_Wl.T), (rm1, rm2_Wr.T)], rm2_bl, True)
    rm3 = _dense([(sm(rm2), rm3_Wl.T), (rm2, rm3_Wr.T)], rm3_bl, True)
    rm4 = _dense([(sm(rm3), rm4_Wl.T), (rm3, rm4_Wr.T)], rm4_bl, False)

    rm4p = jnp.pad(rm4, ((0, 0), (0, 13)))
    agg_rm4 = sm(rm4p)[:, :3]
    h = _dense(
        [(agg_x, sh_Wl[:, :128].T), (agg_rm4, sh_Wl[:, 128:].T),
         (x, sh_Wr[:, :128].T), (rm4, sh_Wr[:, 128:].T)],
        sh_bl, True)

    j1 = _dense([(sm(h), j1_Wl.T), (h, j1_Wr.T)], j1_bl, True)
    j2 = _dense([(sm(j1), j2_Wl.T), (j1, j2_Wr.T)], j2_bl, True)

    rm_ls, jm, lab = _head(rm4, j2, j3_W.T, j3_b)

    lab2 = lab.reshape(n // 2, 2)
    labcat = jnp.concatenate([lab2, lab2], axis=0)
    j = (jm * labcat).reshape(-1)
    return (rm_ls, j)
